# baseline (speedup 1.0000x reference)
import functools

import jax
import jax.numpy as jnp
from jax import lax
from jax.experimental import pallas as pl
from jax.experimental.pallas import tpu as pltpu
from jax.experimental.pallas import tpu_sc as plsc

N = 16384
D = 64
KPAD = 128
NC = 2
NS = 16
NW = NC * NS
CHUNK = N // NW
NB = CHUNK // 128
AW = 16


def _aux_body(emb_ref, aux_ref):
  x = emb_ref[...]
  normsq = jnp.sum(x * x, axis=1, keepdims=True)
  lane = lax.broadcasted_iota(jnp.int32, (x.shape[0], AW), 1)
  aux_ref[...] = jnp.where(lane == 0, normsq,
                           jnp.where(lane == 1, 1.0, 0.0))


def _sc_body(emb_hbm, aux_hbm, lab_hbm, out_sums, out_aux,
             lab_v, emb_v, aux_v, zb64, zb16, sh_sums, sh_aux):
  c = lax.axis_index("c")
  s = lax.axis_index("s")
  wid = s * NC + c
  base = wid * CHUNK

  zv = jnp.zeros((16,), jnp.float32)

  rows = KPAD // NS
  for i in range(rows):
    for j in range(D // 16):
      zb64[i, pl.ds(j * 16, 16)] = zv
    zb16[i, :] = zv
  pltpu.sync_copy(zb64, sh_sums.at[pl.ds(s * rows, rows)])
  pltpu.sync_copy(zb16, sh_aux.at[pl.ds(s * rows, rows)])

  pltpu.sync_copy(emb_hbm.at[pl.ds(base, CHUNK)], emb_v)
  pltpu.sync_copy(aux_hbm.at[pl.ds(base, CHUNK)], aux_v)
  for b in range(NB):
    pltpu.sync_copy(lab_hbm.at[pl.ds(base + b * 128, 128)], lab_v.at[b])

  plsc.subcore_barrier()

  for b in range(NB):
    pltpu.sync_copy(emb_v.at[pl.ds(b * 128, 128)],
                    sh_sums.at[lab_v.at[b]], add=True)
    pltpu.sync_copy(aux_v.at[pl.ds(b * 128, 128)],
                    sh_aux.at[lab_v.at[b]], add=True)

  plsc.subcore_barrier()

  @pl.when(s == 0)
  def _dump():
    pltpu.sync_copy(sh_sums, out_sums.at[c])
    pltpu.sync_copy(sh_aux, out_aux.at[c])


@functools.partial(
    pl.kernel,
    out_type=(
        jax.ShapeDtypeStruct((NC, KPAD, D), jnp.float32),
        jax.ShapeDtypeStruct((NC, KPAD, AW), jnp.float32),
    ),
    mesh=plsc.VectorSubcoreMesh(
        core_axis_name="c", subcore_axis_name="s",
        num_cores=NC, num_subcores=NS),
    compiler_params=pltpu.CompilerParams(use_tc_tiling_on_sc=False),
    scratch_types=[
        pltpu.VMEM((NB, 128), jnp.int32),
        pltpu.VMEM((CHUNK, D), jnp.float32),
        pltpu.VMEM((CHUNK, AW), jnp.float32),
        pltpu.VMEM((KPAD // NS, D), jnp.float32),
        pltpu.VMEM((KPAD // NS, AW), jnp.float32),
        pltpu.VMEM_SHARED((KPAD, D), jnp.float32),
        pltpu.VMEM_SHARED((KPAD, AW), jnp.float32),
    ],
)
def _sc_partials(emb_hbm, aux_hbm, lab_hbm, out_sums, out_aux, *scratch):
  _sc_body(emb_hbm, aux_hbm, lab_hbm, out_sums, out_aux, *scratch)


def _tc_combine_body(sums_ref, aux_ref, out_ref):
  sums = sums_ref[0] + sums_ref[1]
  aux = aux_ref[0] + aux_ref[1]
  sumsq = aux[:, 0:1]
  cnt = aux[:, 1:2]
  normsq = jnp.sum(sums * sums, axis=1, keepdims=True)
  safe = jnp.maximum(cnt, 1.0)
  per_class = (sumsq - normsq / safe) / safe
  present = cnt > 0.0
  n_unique = jnp.sum(present.astype(jnp.float32))
  out_ref[0, 0] = jnp.sum(jnp.where(present, per_class, 0.0)) / n_unique


def kernel(embeddings, labels):
  lab = labels.astype(jnp.int32)
  aux = pl.pallas_call(
      _aux_body,
      out_shape=jax.ShapeDtypeStruct((N, AW), jnp.float32),
  )(embeddings)
  sums_p, aux_p = _sc_partials(embeddings, aux, lab)
  loss = pl.pallas_call(
      _tc_combine_body,
      out_shape=jax.ShapeDtypeStruct((1, 1), jnp.float32),
      out_specs=pl.BlockSpec(memory_space=pltpu.SMEM),
  )(sums_p, aux_p)
  return loss[0, 0]


# speedup vs baseline: 1.2572x; 1.2572x over previous
import functools

import jax
import jax.numpy as jnp
from jax import lax
from jax.experimental import pallas as pl
from jax.experimental.pallas import tpu as pltpu
from jax.experimental.pallas import tpu_sc as plsc

N = 16384
D = 64
KPAD = 128
NC = 2
NS = 16
NW = NC * NS
CHUNK = N // NW
NB = CHUNK // 128
CW = 16


def _sc_body(emb_hbm, lab_hbm, out_sums, out_sq, out_cnt,
             lab_v, emb_v, sq_v, ones_v, zb64, zb16,
             sh_sums, sh_sq, sh_cnt):
  c = lax.axis_index("c")
  s = lax.axis_index("s")
  wid = s * NC + c
  base = wid * CHUNK

  zv = jnp.zeros((16,), jnp.float32)
  ov = jnp.ones((16,), jnp.float32)

  rows = KPAD // NS
  for i in range(rows):
    for j in range(D // 16):
      zb64[i, pl.ds(j * 16, 16)] = zv
    zb16[i, :] = zv
  pltpu.sync_copy(zb64, sh_sums.at[pl.ds(s * rows, rows)])
  pltpu.sync_copy(zb64, sh_sq.at[pl.ds(s * rows, rows)])
  pltpu.sync_copy(zb16, sh_cnt.at[pl.ds(s * rows, rows)])

  def ones_body(i, carry):
    ones_v[i, :] = ov
    return carry
  lax.fori_loop(0, 128, ones_body, 0)

  pltpu.sync_copy(emb_hbm.at[pl.ds(base, CHUNK)], emb_v)
  for b in range(NB):
    pltpu.sync_copy(lab_hbm.at[pl.ds(base + b * 128, 128)], lab_v.at[b])

  plsc.subcore_barrier()

  for b in range(NB):
    @functools.partial(plsc.parallel_loop, 0, 128, unroll=4)
    def row_body(r):
      for j in range(D // 16):
        v = emb_v[b * 128 + r, pl.ds(j * 16, 16)]
        sq_v[r, pl.ds(j * 16, 16)] = v * v

    pltpu.sync_copy(emb_v.at[pl.ds(b * 128, 128)],
                    sh_sums.at[lab_v.at[b]], add=True)
    pltpu.sync_copy(sq_v, sh_sq.at[lab_v.at[b]], add=True)
    pltpu.sync_copy(ones_v, sh_cnt.at[lab_v.at[b]], add=True)

  plsc.subcore_barrier()

  @pl.when(s == 0)
  def _dump():
    pltpu.sync_copy(sh_sums, out_sums.at[c])
    pltpu.sync_copy(sh_sq, out_sq.at[c])
    pltpu.sync_copy(sh_cnt, out_cnt.at[c])


@functools.partial(
    pl.kernel,
    out_type=(
        jax.ShapeDtypeStruct((NC, KPAD, D), jnp.float32),
        jax.ShapeDtypeStruct((NC, KPAD, D), jnp.float32),
        jax.ShapeDtypeStruct((NC, KPAD, CW), jnp.float32),
    ),
    mesh=plsc.VectorSubcoreMesh(
        core_axis_name="c", subcore_axis_name="s",
        num_cores=NC, num_subcores=NS),
    compiler_params=pltpu.CompilerParams(use_tc_tiling_on_sc=False),
    scratch_types=[
        pltpu.VMEM((NB, 128), jnp.int32),
        pltpu.VMEM((CHUNK, D), jnp.float32),
        pltpu.VMEM((128, D), jnp.float32),
        pltpu.VMEM((128, CW), jnp.float32),
        pltpu.VMEM((KPAD // NS, D), jnp.float32),
        pltpu.VMEM((KPAD // NS, CW), jnp.float32),
        pltpu.VMEM_SHARED((KPAD, D), jnp.float32),
        pltpu.VMEM_SHARED((KPAD, D), jnp.float32),
        pltpu.VMEM_SHARED((KPAD, CW), jnp.float32),
    ],
)
def _sc_partials(emb_hbm, lab_hbm, out_sums, out_sq, out_cnt, *scratch):
  _sc_body(emb_hbm, lab_hbm, out_sums, out_sq, out_cnt, *scratch)


def _tc_combine_body(sums_ref, sq_ref, cnt_ref, out_ref):
  sums = sums_ref[0] + sums_ref[1]
  sq = sq_ref[0] + sq_ref[1]
  cnt = cnt_ref[0, :, 0:1] + cnt_ref[1, :, 0:1]
  sumsq = jnp.sum(sq, axis=1, keepdims=True)
  normsq = jnp.sum(sums * sums, axis=1, keepdims=True)
  safe = jnp.maximum(cnt, 1.0)
  per_class = (sumsq - normsq / safe) / safe
  present = cnt > 0.0
  n_unique = jnp.sum(present.astype(jnp.float32))
  out_ref[0, 0] = jnp.sum(jnp.where(present, per_class, 0.0)) / n_unique


def kernel(embeddings, labels):
  lab = labels.astype(jnp.int32)
  sums_p, sq_p, cnt_p = _sc_partials(embeddings, lab)
  loss = pl.pallas_call(
      _tc_combine_body,
      out_shape=jax.ShapeDtypeStruct((1, 1), jnp.float32),
      out_specs=pl.BlockSpec(memory_space=pltpu.SMEM),
  )(sums_p, sq_p, cnt_p)
  return loss[0, 0]
